# Initial kernel scaffold; baseline (speedup 1.0000x reference)
import jax
import jax.numpy as jnp
from jax.experimental import pallas as pl


def kernel(input_ids, wte_weight, soft_prompt):
    raise NotImplementedError("write your pallas kernel here")


# speedup vs baseline: 1.0401x; 1.0401x over previous
import jax
import jax.numpy as jnp
from jax import lax
from jax.experimental import pallas as pl
from jax.experimental.pallas import tpu as pltpu
from jax.experimental.pallas import tpu_sc as plsc

VOCAB = 100000
D = 1024
NT = 100
B = 4
S = 2048
TOT = NT + S

NC, NS = 2, 16
NW = NC * NS
WPB = NW // B
ROWS_PER_W = (B * S) // NW
CHUNK = 32
NCHUNK = ROWS_PER_W // CHUNK
L = 16


def _sc_body(ids_hbm, wte_hbm, sp_hbm, out_hbm, idx_v, gbuf0, gbuf1, spbuf,
             oidx0, oidx1, g0, g1, s0, s1):
    c = lax.axis_index("c")
    s = lax.axis_index("s")
    wid = s * NC + c
    b = wid // WPB
    q = wid % WPB
    base_id = wid * ROWS_PER_W
    out_base = b * TOT + NT + q * ROWS_PER_W
    iota = lax.iota(jnp.int32, L)

    pltpu.sync_copy(ids_hbm.at[pl.ds(base_id, ROWS_PER_W)], idx_v)

    @pl.when(q < 3)
    def _sp_main():
        pltpu.sync_copy(sp_hbm.at[pl.ds(q * CHUNK, CHUNK)], gbuf0)
        row0 = b * TOT + q * CHUNK
        oidx0[pl.ds(0, L)] = row0 + iota
        oidx0[pl.ds(L, L)] = row0 + L + iota
        pltpu.async_copy(gbuf0, out_hbm.at[oidx0], s0).wait()

    @pl.when(q == 3)
    def _sp_rem():
        pltpu.async_copy(sp_hbm.at[(NT - L) + iota], spbuf, g0).wait()
        pltpu.async_copy(spbuf, out_hbm.at[b * TOT + (NT - L) + iota],
                         s0).wait()

    gbufs = (gbuf0, gbuf1)
    gsems = (g0, g1)
    oidxs = (oidx0, oidx1)
    ssems = (s0, s1)

    def start_gather(k):
        return pltpu.async_copy(
            wte_hbm.at[idx_v.at[pl.ds(k * CHUNK, CHUNK)]],
            gbufs[k % 2], gsems[k % 2])

    def fill_oidx(k):
        o = oidxs[k % 2]
        row0 = out_base + k * CHUNK
        o[pl.ds(0, L)] = row0 + iota
        o[pl.ds(L, L)] = row0 + L + iota

    def start_scatter(k):
        return pltpu.async_copy(gbufs[k % 2], out_hbm.at[oidxs[k % 2]],
                                ssems[k % 2])

    gh = [None] * NCHUNK
    sh = [None] * NCHUNK
    gh[0] = start_gather(0)
    for k in range(NCHUNK):
        if k + 1 < NCHUNK:
            if k - 1 >= 0:
                sh[k - 1].wait()
            gh[k + 1] = start_gather(k + 1)
        gh[k].wait()
        fill_oidx(k)
        sh[k] = start_scatter(k)
    sh[NCHUNK - 2].wait()
    sh[NCHUNK - 1].wait()


@jax.jit
def kernel(input_ids, wte_weight, soft_prompt):
    ids_flat = input_ids.reshape(B * S).astype(jnp.int32)
    mesh = plsc.VectorSubcoreMesh(core_axis_name="c", subcore_axis_name="s",
                                  num_cores=NC, num_subcores=NS)
    out = pl.kernel(
        _sc_body,
        out_type=jax.ShapeDtypeStruct((B * TOT, D), jnp.float32),
        mesh=mesh,
        scratch_types=[
            pltpu.VMEM((ROWS_PER_W,), jnp.int32),
            pltpu.VMEM((CHUNK, D), jnp.float32),
            pltpu.VMEM((CHUNK, D), jnp.float32),
            pltpu.VMEM((L, D), jnp.float32),
            pltpu.VMEM((CHUNK,), jnp.int32),
            pltpu.VMEM((CHUNK,), jnp.int32),
            pltpu.SemaphoreType.DMA,
            pltpu.SemaphoreType.DMA,
            pltpu.SemaphoreType.DMA,
            pltpu.SemaphoreType.DMA,
        ],
    )(ids_flat, wte_weight, soft_prompt)
    return out.reshape(B, TOT, D)


# speedup vs baseline: 1.0832x; 1.0414x over previous
import jax
import jax.numpy as jnp
from jax import lax
from jax.experimental import pallas as pl
from jax.experimental.pallas import tpu as pltpu
from jax.experimental.pallas import tpu_sc as plsc

VOCAB = 100000
D = 1024
NT = 100
B = 4
S = 2048
TOT = NT + S

NC, NS = 2, 16
NW = NC * NS
WPB = NW // B
ROWS_PER_W = (B * S) // NW
CHUNK = 32
NCHUNK = ROWS_PER_W // CHUNK
L = 16


def _sc_body(ids_hbm, wte_hbm, sp_hbm, out_hbm, idx_v, gbuf0, gbuf1, spbuf,
             oidx0, oidx1, g0, g1, s0, s1):
    c = lax.axis_index("c")
    s = lax.axis_index("s")
    wid = s * NC + c
    b = wid // WPB
    q = wid % WPB
    base_id = wid * ROWS_PER_W
    out_base = NT + q * ROWS_PER_W
    iota = lax.iota(jnp.int32, L)

    pltpu.sync_copy(ids_hbm.at[pl.ds(base_id, ROWS_PER_W)], idx_v)

    @pl.when(q < 3)
    def _sp_main():
        pltpu.sync_copy(sp_hbm.at[pl.ds(q * CHUNK, CHUNK)], gbuf0)
        row0 = q * CHUNK
        oidx0[pl.ds(0, L)] = row0 + iota
        oidx0[pl.ds(L, L)] = row0 + L + iota
        pltpu.async_copy(gbuf0, out_hbm.at[b].at[oidx0], s0).wait()

    @pl.when(q == 3)
    def _sp_rem():
        pltpu.async_copy(sp_hbm.at[(NT - L) + iota], spbuf, g0).wait()
        pltpu.async_copy(spbuf, out_hbm.at[b].at[(NT - L) + iota],
                         s0).wait()

    gbufs = (gbuf0, gbuf1)
    gsems = (g0, g1)
    oidxs = (oidx0, oidx1)
    ssems = (s0, s1)

    def start_gather(k):
        return pltpu.async_copy(
            wte_hbm.at[idx_v.at[pl.ds(k * CHUNK, CHUNK)]],
            gbufs[k % 2], gsems[k % 2])

    def fill_oidx(k):
        o = oidxs[k % 2]
        row0 = out_base + k * CHUNK
        o[pl.ds(0, L)] = row0 + iota
        o[pl.ds(L, L)] = row0 + L + iota

    def start_scatter(k):
        return pltpu.async_copy(gbufs[k % 2],
                                out_hbm.at[b].at[oidxs[k % 2]],
                                ssems[k % 2])

    gh = [None] * NCHUNK
    sh = [None] * NCHUNK
    gh[0] = start_gather(0)
    for k in range(NCHUNK):
        if k + 1 < NCHUNK:
            if k - 1 >= 0:
                sh[k - 1].wait()
            gh[k + 1] = start_gather(k + 1)
        gh[k].wait()
        fill_oidx(k)
        sh[k] = start_scatter(k)
    sh[NCHUNK - 2].wait()
    sh[NCHUNK - 1].wait()


@jax.jit
def kernel(input_ids, wte_weight, soft_prompt):
    ids_flat = input_ids.reshape(B * S).astype(jnp.int32)
    mesh = plsc.VectorSubcoreMesh(core_axis_name="c", subcore_axis_name="s",
                                  num_cores=NC, num_subcores=NS)
    out = pl.kernel(
        _sc_body,
        out_type=jax.ShapeDtypeStruct((B, TOT, D), jnp.float32),
        mesh=mesh,
        scratch_types=[
            pltpu.VMEM((ROWS_PER_W,), jnp.int32),
            pltpu.VMEM((CHUNK, D), jnp.float32),
            pltpu.VMEM((CHUNK, D), jnp.float32),
            pltpu.VMEM((L, D), jnp.float32),
            pltpu.VMEM((CHUNK,), jnp.int32),
            pltpu.VMEM((CHUNK,), jnp.int32),
            pltpu.SemaphoreType.DMA,
            pltpu.SemaphoreType.DMA,
            pltpu.SemaphoreType.DMA,
            pltpu.SemaphoreType.DMA,
        ],
    )(ids_flat, wte_weight, soft_prompt)
    return out


# speedup vs baseline: 1.1984x; 1.1064x over previous
import jax
import jax.numpy as jnp
from jax import lax
from jax.experimental import pallas as pl
from jax.experimental.pallas import tpu as pltpu
from jax.experimental.pallas import tpu_sc as plsc

VOCAB = 100000
D = 1024
NT = 100
B = 4
S = 2048
TOT = NT + S

NC, NS = 2, 16
NW = NC * NS
WPB = NW // B
ROWS_PER_W = (B * S) // NW
L = 16
NG = ROWS_PER_W // L
SP_SLABS = NT * 8 // WPB


def _sc_body(ids_hbm, wte_hbm, sp_hbm, out_hbm,
             idsbuf, idx_v, gbuf0, gbuf1, spbuf,
             gidx0, gidx1, sidx0, sidx1, spgidx, spsidx,
             g0, g1, s0, s1, idsem, spg, sps):
    c = lax.axis_index("c")
    s = lax.axis_index("s")
    wid = s * NC + c
    b = wid // WPB
    q = wid % WPB
    iota = lax.iota(jnp.int32, L)

    for u in range(8):
        m = u * L + iota
        mm = jnp.where(m < SP_SLABS, m, m - SP_SLABS)
        g = q * SP_SLABS + mm
        tt = g >> 3
        dcv = g & 7
        spgidx[pl.ds(u * L, L)] = ((tt >> 3) << 6) + (dcv << 3) + (tt & 7)
        spsidx[pl.ds(u * L, L)] = tt * 32 + b * 8 + dcv
    sp_gather = pltpu.async_copy(sp_hbm.at[spgidx], spbuf, spg)

    pltpu.async_copy(ids_hbm.at[WPB * q + b + 4 * (iota & 1)],
                     idsbuf, idsem).wait()
    for m in range(16):
        idx_v[pl.ds(m * L, L)] = idsbuf[m // 8, pl.ds((m % 8) * L, L)]

    sp_gather.wait()
    sp_scatter = pltpu.async_copy(spbuf, out_hbm.at[spsidx], sps)

    gbufs = (gbuf0, gbuf1)
    gidxs = (gidx0, gidx1)
    sidxs = (sidx0, sidx1)
    gsems = (g0, g1)
    ssems = (s0, s1)
    t_base = NT + q * ROWS_PER_W

    def body(i, carry):
        for par in range(2):
            g = 2 * i + par

            @pl.when(i > 0)
            def _free_buf():
                pltpu.make_async_copy(gbufs[par], out_hbm.at[sidxs[par]],
                                      ssems[par]).wait()

            v = idx_v[pl.ds(g * L, L)]
            gbase = ((v >> 3) << 6) + (v & 7)
            for dc in range(8):
                gidxs[par][pl.ds(dc * L, L)] = gbase + dc * 8
            pltpu.async_copy(wte_hbm.at[gidxs[par]], gbufs[par], gsems[par])
        for par in range(2):
            g = 2 * i + par
            pltpu.make_async_copy(wte_hbm.at[gidxs[par]], gbufs[par],
                                  gsems[par]).wait()
            sbase = (t_base + g * L + iota) * 32 + b * 8
            for dc in range(8):
                sidxs[par][pl.ds(dc * L, L)] = sbase + dc
            pltpu.async_copy(gbufs[par], out_hbm.at[sidxs[par]], ssems[par])
        return carry

    lax.fori_loop(0, NG // 2, body, 0)

    sp_scatter.wait()
    for par in range(2):
        pltpu.make_async_copy(gbufs[par], out_hbm.at[sidxs[par]],
                              ssems[par]).wait()


@jax.jit
def kernel(input_ids, wte_weight, soft_prompt):
    ids_view = (input_ids.astype(jnp.int32)
                .reshape(B, S // 128, 128).transpose(1, 0, 2)
                .reshape(B * S // 128, 128))
    wte_view = (wte_weight.reshape(VOCAB // 8, 8, 8, 128)
                .transpose(0, 2, 1, 3).reshape(VOCAB * 8, 128))
    sp_pad = jnp.pad(soft_prompt, ((0, 4), (0, 0)))
    sp_view = (sp_pad.reshape(13, 8, 8, 128)
               .transpose(0, 2, 1, 3).reshape(13 * 64, 128))

    mesh = plsc.VectorSubcoreMesh(core_axis_name="c", subcore_axis_name="s",
                                  num_cores=NC, num_subcores=NS)
    out = pl.kernel(
        _sc_body,
        out_type=jax.ShapeDtypeStruct((B * TOT * 8, 128), jnp.float32),
        mesh=mesh,
        scratch_types=[
            pltpu.VMEM((16, 128), jnp.int32),
            pltpu.VMEM((ROWS_PER_W,), jnp.int32),
            pltpu.VMEM((128, 128), jnp.float32),
            pltpu.VMEM((128, 128), jnp.float32),
            pltpu.VMEM((128, 128), jnp.float32),
            pltpu.VMEM((128,), jnp.int32),
            pltpu.VMEM((128,), jnp.int32),
            pltpu.VMEM((128,), jnp.int32),
            pltpu.VMEM((128,), jnp.int32),
            pltpu.VMEM((128,), jnp.int32),
            pltpu.VMEM((128,), jnp.int32),
            pltpu.SemaphoreType.DMA,
            pltpu.SemaphoreType.DMA,
            pltpu.SemaphoreType.DMA,
            pltpu.SemaphoreType.DMA,
            pltpu.SemaphoreType.DMA,
            pltpu.SemaphoreType.DMA,
            pltpu.SemaphoreType.DMA,
        ],
    )(ids_view, wte_view, sp_view)
    return (out.reshape(TOT, B, 8, 128).transpose(1, 0, 2, 3)
            .reshape(B, TOT, D))


# speedup vs baseline: 2.1732x; 1.8134x over previous
import jax
import jax.numpy as jnp
from jax import lax
from jax.experimental import pallas as pl
from jax.experimental.pallas import tpu as pltpu
from jax.experimental.pallas import tpu_sc as plsc

VOCAB = 100000
D = 1024
NT = 100
B = 4
S = 2048
TOT = NT + S

NC, NS = 2, 16
NW = NC * NS
WPB = NW // B
ROWS_PER_W = (B * S) // NW
L = 16
NG = ROWS_PER_W // L
SP_SLABS = NT * 8 // WPB


def _sc_body(ids_hbm, wte_hbm, sp_hbm, out_hbm,
             idsbuf, idx_v, gbuf0, gbuf1, spbuf,
             gidx0, gidx1, sidx0, sidx1, spgidx, spsidx,
             g0, g1, s0, s1, idsem, spg, sps):
    c = lax.axis_index("c")
    s = lax.axis_index("s")
    wid = s * NC + c
    b = wid // WPB
    q = wid % WPB
    iota = lax.iota(jnp.int32, L)

    for u in range(8):
        m = u * L + iota
        mm = jnp.where(m < SP_SLABS, m, m - SP_SLABS)
        g = q * SP_SLABS + mm
        tt = g >> 3
        dcv = g & 7
        spgidx[pl.ds(u * L, L)] = ((tt >> 3) << 6) + (dcv << 3) + (tt & 7)
        spsidx[pl.ds(u * L, L)] = tt * 32 + dcv * 4 + b
    sp_gather = pltpu.async_copy(sp_hbm.at[spgidx], spbuf, spg)

    pltpu.async_copy(ids_hbm.at[WPB * q + b + 4 * (iota & 1)],
                     idsbuf, idsem).wait()
    for m in range(16):
        idx_v[pl.ds(m * L, L)] = idsbuf[m // 8, pl.ds((m % 8) * L, L)]

    sp_gather.wait()
    sp_scatter = pltpu.async_copy(spbuf, out_hbm.at[spsidx], sps)

    gbufs = (gbuf0, gbuf1)
    gidxs = (gidx0, gidx1)
    sidxs = (sidx0, sidx1)
    gsems = (g0, g1)
    ssems = (s0, s1)
    t_base = NT + q * ROWS_PER_W

    def body(i, carry):
        for par in range(2):
            g = 2 * i + par

            @pl.when(i > 0)
            def _free_buf():
                pltpu.make_async_copy(gbufs[par], out_hbm.at[sidxs[par]],
                                      ssems[par]).wait()

            v = idx_v[pl.ds(g * L, L)]
            gbase = ((v >> 3) << 6) + (v & 7)
            for dc in range(8):
                gidxs[par][pl.ds(dc * L, L)] = gbase + dc * 8
            pltpu.async_copy(wte_hbm.at[gidxs[par]], gbufs[par], gsems[par])
        for par in range(2):
            g = 2 * i + par
            pltpu.make_async_copy(wte_hbm.at[gidxs[par]], gbufs[par],
                                  gsems[par]).wait()
            sbase = (t_base + g * L + iota) * 32 + b
            for dc in range(8):
                sidxs[par][pl.ds(dc * L, L)] = sbase + dc * 4
            pltpu.async_copy(gbufs[par], out_hbm.at[sidxs[par]], ssems[par])
        return carry

    lax.fori_loop(0, NG // 2, body, 0)

    sp_scatter.wait()
    for par in range(2):
        pltpu.make_async_copy(gbufs[par], out_hbm.at[sidxs[par]],
                              ssems[par]).wait()


@jax.jit
def kernel(input_ids, wte_weight, soft_prompt):
    ids_view = (input_ids.astype(jnp.int32)
                .reshape(B, S // 128, 128).transpose(1, 0, 2)
                .reshape(B * S // 128, 128))
    wte_view = (wte_weight.reshape(VOCAB // 8, 8, 8, 128)
                .transpose(0, 2, 1, 3).reshape(VOCAB * 8, 128))
    sp_pad = jnp.pad(soft_prompt, ((0, 4), (0, 0)))
    sp_view = (sp_pad.reshape(13, 8, 8, 128)
               .transpose(0, 2, 1, 3).reshape(13 * 64, 128))

    mesh = plsc.VectorSubcoreMesh(core_axis_name="c", subcore_axis_name="s",
                                  num_cores=NC, num_subcores=NS)
    out = pl.kernel(
        _sc_body,
        out_type=jax.ShapeDtypeStruct((B * TOT * 8, 128), jnp.float32),
        mesh=mesh,
        scratch_types=[
            pltpu.VMEM((16, 128), jnp.int32),
            pltpu.VMEM((ROWS_PER_W,), jnp.int32),
            pltpu.VMEM((128, 128), jnp.float32),
            pltpu.VMEM((128, 128), jnp.float32),
            pltpu.VMEM((128, 128), jnp.float32),
            pltpu.VMEM((128,), jnp.int32),
            pltpu.VMEM((128,), jnp.int32),
            pltpu.VMEM((128,), jnp.int32),
            pltpu.VMEM((128,), jnp.int32),
            pltpu.VMEM((128,), jnp.int32),
            pltpu.VMEM((128,), jnp.int32),
            pltpu.SemaphoreType.DMA,
            pltpu.SemaphoreType.DMA,
            pltpu.SemaphoreType.DMA,
            pltpu.SemaphoreType.DMA,
            pltpu.SemaphoreType.DMA,
            pltpu.SemaphoreType.DMA,
            pltpu.SemaphoreType.DMA,
        ],
    )(ids_view, wte_view, sp_view)
    return (out.reshape(TOT, 8, B, 128).transpose(2, 0, 1, 3)
            .reshape(B, TOT, D))


# speedup vs baseline: 2.2571x; 1.0386x over previous
import jax
import jax.numpy as jnp
from jax import lax
from jax.experimental import pallas as pl
from jax.experimental.pallas import tpu as pltpu
from jax.experimental.pallas import tpu_sc as plsc

VOCAB = 100000
D = 1024
NT = 100
B = 4
S = 2048
TOT = NT + S

NC, NS = 2, 16
NW = NC * NS
WPB = NW // B
ROWS_PER_W = (B * S) // NW
L = 16
NG = ROWS_PER_W // L
SP_SLABS = NT * 8 // WPB


NBUF = 4


def _sc_body(ids_hbm, wte_hbm, sp_hbm, out_hbm,
             idsbuf, idx_v, gbuf0, gbuf1, gbuf2, gbuf3, spbuf,
             gidx0, gidx1, gidx2, gidx3, sidx0, sidx1, sidx2, sidx3,
             spgidx, spsidx,
             g0, g1, g2, g3, s0, s1, s2, s3, idsem, spg, sps):
    c = lax.axis_index("c")
    s = lax.axis_index("s")
    wid = s * NC + c
    b = wid // WPB
    q = wid % WPB
    iota = lax.iota(jnp.int32, L)

    for u in range(8):
        m = u * L + iota
        mm = jnp.where(m < SP_SLABS, m, m - SP_SLABS)
        g = q * SP_SLABS + mm
        tt = g >> 3
        dcv = g & 7
        spgidx[pl.ds(u * L, L)] = ((tt >> 3) << 6) + (dcv << 3) + (tt & 7)
        spsidx[pl.ds(u * L, L)] = tt * 32 + dcv * 4 + b
    sp_gather = pltpu.async_copy(sp_hbm.at[spgidx], spbuf, spg)

    pltpu.async_copy(ids_hbm.at[WPB * q + b + 4 * (iota & 1)],
                     idsbuf, idsem).wait()
    for m in range(16):
        idx_v[pl.ds(m * L, L)] = idsbuf[m // 8, pl.ds((m % 8) * L, L)]

    sp_gather.wait()
    sp_scatter = pltpu.async_copy(spbuf, out_hbm.at[spsidx], sps)

    gbufs = (gbuf0, gbuf1, gbuf2, gbuf3)
    gidxs = (gidx0, gidx1, gidx2, gidx3)
    sidxs = (sidx0, sidx1, sidx2, sidx3)
    gsems = (g0, g1, g2, g3)
    ssems = (s0, s1, s2, s3)
    t_base = NT + q * ROWS_PER_W

    def body(i, carry):
        for par in range(NBUF):
            g = NBUF * i + par

            @pl.when(i > 0)
            def _free_buf():
                pltpu.make_async_copy(gbufs[par], out_hbm.at[sidxs[par]],
                                      ssems[par]).wait()

            v = idx_v[pl.ds(g * L, L)]
            gbase = ((v >> 3) << 6) + (v & 7)
            for dc in range(8):
                gidxs[par][pl.ds(dc * L, L)] = gbase + dc * 8
            pltpu.async_copy(wte_hbm.at[gidxs[par]], gbufs[par], gsems[par])
        for par in range(NBUF):
            g = NBUF * i + par
            pltpu.make_async_copy(wte_hbm.at[gidxs[par]], gbufs[par],
                                  gsems[par]).wait()
            sbase = (t_base + g * L + iota) * 32 + b
            for dc in range(8):
                sidxs[par][pl.ds(dc * L, L)] = sbase + dc * 4
            pltpu.async_copy(gbufs[par], out_hbm.at[sidxs[par]], ssems[par])
        return carry

    lax.fori_loop(0, NG // NBUF, body, 0)

    sp_scatter.wait()
    for par in range(NBUF):
        pltpu.make_async_copy(gbufs[par], out_hbm.at[sidxs[par]],
                              ssems[par]).wait()


@jax.jit
def kernel(input_ids, wte_weight, soft_prompt):
    ids_view = (input_ids.astype(jnp.int32)
                .reshape(B, S // 128, 128).transpose(1, 0, 2)
                .reshape(B * S // 128, 128))
    wte_view = (wte_weight.reshape(VOCAB // 8, 8, 8, 128)
                .transpose(0, 2, 1, 3).reshape(VOCAB * 8, 128))
    sp_pad = jnp.pad(soft_prompt, ((0, 4), (0, 0)))
    sp_view = (sp_pad.reshape(13, 8, 8, 128)
               .transpose(0, 2, 1, 3).reshape(13 * 64, 128))

    mesh = plsc.VectorSubcoreMesh(core_axis_name="c", subcore_axis_name="s",
                                  num_cores=NC, num_subcores=NS)
    out = pl.kernel(
        _sc_body,
        out_type=jax.ShapeDtypeStruct((B * TOT * 8, 128), jnp.float32),
        mesh=mesh,
        scratch_types=[
            pltpu.VMEM((16, 128), jnp.int32),
            pltpu.VMEM((ROWS_PER_W,), jnp.int32),
            pltpu.VMEM((128, 128), jnp.float32),
            pltpu.VMEM((128, 128), jnp.float32),
            pltpu.VMEM((128, 128), jnp.float32),
            pltpu.VMEM((128, 128), jnp.float32),
            pltpu.VMEM((128, 128), jnp.float32),
            pltpu.VMEM((128,), jnp.int32),
            pltpu.VMEM((128,), jnp.int32),
            pltpu.VMEM((128,), jnp.int32),
            pltpu.VMEM((128,), jnp.int32),
            pltpu.VMEM((128,), jnp.int32),
            pltpu.VMEM((128,), jnp.int32),
            pltpu.VMEM((128,), jnp.int32),
            pltpu.VMEM((128,), jnp.int32),
            pltpu.VMEM((128,), jnp.int32),
            pltpu.VMEM((128,), jnp.int32),
            pltpu.SemaphoreType.DMA,
            pltpu.SemaphoreType.DMA,
            pltpu.SemaphoreType.DMA,
            pltpu.SemaphoreType.DMA,
            pltpu.SemaphoreType.DMA,
            pltpu.SemaphoreType.DMA,
            pltpu.SemaphoreType.DMA,
            pltpu.SemaphoreType.DMA,
            pltpu.SemaphoreType.DMA,
            pltpu.SemaphoreType.DMA,
            pltpu.SemaphoreType.DMA,
        ],
    )(ids_view, wte_view, sp_view)
    return (out.reshape(TOT, 8, B, 128).transpose(2, 0, 1, 3)
            .reshape(B, TOT, D))
